# baseline (speedup 1.0000x reference)
import functools

import jax
import jax.numpy as jnp
from jax import lax
from jax.experimental import pallas as pl
from jax.experimental.pallas import tpu as pltpu
from jax.experimental.pallas import tpu_sc as plsc

B, C, H, W = 64, 384, 28, 28
D = H * W
ROWS = B * C
NC, NS, L = 2, 16, 16
NW = NC * NS
RPW = ROWS // NW
BPW = RPW // C
GROUPS = C // L
CHUNK = 48
NCHUNK = RPW // CHUNK
NBUF = 3

_mesh = plsc.VectorSubcoreMesh(
    core_axis_name="c", subcore_axis_name="s", num_cores=NC, num_subcores=NS
)


@functools.partial(
    pl.kernel,
    out_type=jax.ShapeDtypeStruct((ROWS, D), jnp.float32),
    mesh=_mesh,
    scratch_types=[
        pltpu.VMEM((C,), jnp.int32),
        pltpu.VMEM((NCHUNK, CHUNK), jnp.int32),
        pltpu.VMEM((NBUF, CHUNK, D), jnp.float32),
        [pltpu.SemaphoreType.DMA] * NBUF,
        [pltpu.SemaphoreType.DMA] * NBUF,
    ],
    compiler_params=pltpu.CompilerParams(use_tc_tiling_on_sc=False),
)
def _prune_gather(x_hbm, idx_hbm, out_hbm, cidx_v, rows_v, buf_v, gsems, ssems):
    wid = lax.axis_index("s") * NC + lax.axis_index("c")
    base_row = wid * RPW

    pltpu.sync_copy(idx_hbm, cidx_v)

    for b in range(BPW):
        boff = base_row + b * C
        for g in range(GROUPS):
            vec = cidx_v[pl.ds(g * L, L)] + boff
            r = b * C + g * L
            rows_v[r // CHUNK, pl.ds(r % CHUNK, L)] = vec

    def gather(ci, slot):
        return pltpu.async_copy(
            x_hbm.at[rows_v.at[ci]], buf_v.at[slot], gsems[slot]
        )

    def scatter(ci, slot):
        return pltpu.async_copy(
            buf_v.at[slot],
            out_hbm.at[pl.ds(base_row + ci * CHUNK, CHUNK)],
            ssems[slot],
        )

    gh = [None] * NCHUNK
    sh = [None] * NCHUNK
    for ci in range(min(NBUF, NCHUNK)):
        gh[ci] = gather(ci, ci)
    for ci in range(NCHUNK):
        slot = ci % NBUF
        gh[ci].wait()
        nxt = ci + NBUF - 1
        if 1 <= ci and nxt < NCHUNK:
            pslot = (ci - 1) % NBUF
            sh[ci - 1].wait()
            gh[nxt] = gather(nxt, pslot)
        sh[ci] = scatter(ci, slot)
    for ci in range(max(0, NCHUNK - NBUF), NCHUNK):
        sh[ci].wait()


@jax.jit
def kernel(x, indices):
    out = _prune_gather(x.reshape(ROWS, D), indices)
    return out.reshape(B, C, H, W)


# speedup vs baseline: 3.4683x; 3.4683x over previous
import functools

import jax
import jax.numpy as jnp
from jax import lax
from jax.experimental import pallas as pl
from jax.experimental.pallas import tpu as pltpu
from jax.experimental.pallas import tpu_sc as plsc

B, C, H, W = 64, 384, 28, 28
L = 16
NC, NS = 2, 16
NW = NC * NS
PR = H * W * B
RPW = PR // NW
CH = 56
NCH = RPW // CH
GROUPS = C // L

_mesh = plsc.VectorSubcoreMesh(
    core_axis_name="c", subcore_axis_name="s", num_cores=NC, num_subcores=NS
)


@functools.partial(
    pl.kernel,
    out_type=jax.ShapeDtypeStruct((PR, C), jnp.float32),
    mesh=_mesh,
    scratch_types=[
        pltpu.VMEM((C,), jnp.int32),
        [pltpu.VMEM((CH, C), jnp.float32)] * 2,
        [pltpu.VMEM((CH, C), jnp.float32)] * 2,
        [pltpu.SemaphoreType.DMA] * 2,
        [pltpu.SemaphoreType.DMA] * 2,
    ],
    compiler_params=pltpu.CompilerParams(needs_layout_passes=False),
)
def _lane_gather(x_hbm, idx_hbm, out_hbm, cidx_v, inbufs, outbufs, isems, osems):
    wid = lax.axis_index("s") * NC + lax.axis_index("c")
    base = wid * RPW

    pltpu.sync_copy(idx_hbm, cidx_v)
    colvs = [cidx_v[pl.ds(g * L, L)] for g in range(GROUPS)]

    def din(ci, slot):
        return pltpu.async_copy(
            x_hbm.at[pl.ds(base + ci * CH, CH)], inbufs[slot], isems[slot]
        )

    def dout(ci, slot):
        return pltpu.async_copy(
            outbufs[slot], out_hbm.at[pl.ds(base + ci * CH, CH)], osems[slot]
        )

    ih = [None] * NCH
    oh = [None] * NCH
    ih[0] = din(0, 0)
    ih[1] = din(1, 1)
    for ci in range(NCH):
        slot = ci % 2
        ih[ci].wait()
        if ci >= 2:
            oh[ci - 2].wait()
        ib, ob = inbufs[slot], outbufs[slot]

        @pl.loop(0, CH)
        def _row(r):
            rowv = jnp.full((L,), r, jnp.int32)
            for g in range(GROUPS):
                ob[r, pl.ds(g * L, L)] = plsc.load_gather(ib, [rowv, colvs[g]])

        oh[ci] = dout(ci, slot)
        if ci + 2 < NCH:
            ih[ci + 2] = din(ci + 2, slot)
    oh[NCH - 2].wait()
    oh[NCH - 1].wait()


@jax.jit
def kernel(x, indices):
    xt = jnp.transpose(x, (2, 3, 0, 1)).reshape(PR, C)
    out = _lane_gather(xt, indices)
    return jnp.transpose(out.reshape(H, W, B, C), (2, 3, 0, 1))


# speedup vs baseline: 3.7068x; 1.0688x over previous
import functools

import jax
import jax.numpy as jnp
from jax import lax
from jax.experimental import pallas as pl
from jax.experimental.pallas import tpu as pltpu
from jax.experimental.pallas import tpu_sc as plsc

B, C, H, W = 64, 384, 28, 28
L = 16
NC, NS = 2, 16
NW = NC * NS
PR = H * W * B
RPW = PR // NW
CH = 56
NCH = RPW // CH
GROUPS = C // L

_mesh = plsc.VectorSubcoreMesh(
    core_axis_name="c", subcore_axis_name="s", num_cores=NC, num_subcores=NS
)


@functools.partial(
    pl.kernel,
    out_type=jax.ShapeDtypeStruct((PR, C), jnp.float32),
    mesh=_mesh,
    scratch_types=[
        pltpu.VMEM((C,), jnp.int32),
        [pltpu.VMEM((CH, C), jnp.float32)] * 2,
        [pltpu.VMEM((CH, C), jnp.float32)] * 2,
        [pltpu.SemaphoreType.DMA] * 2,
        [pltpu.SemaphoreType.DMA] * 2,
    ],
    compiler_params=pltpu.CompilerParams(needs_layout_passes=False),
)
def _lane_gather(x_hbm, idx_hbm, out_hbm, cidx_v, inbufs, outbufs, isems, osems):
    wid = lax.axis_index("s") * NC + lax.axis_index("c")
    base = wid * RPW

    pltpu.sync_copy(idx_hbm, cidx_v)
    colvs = [cidx_v[pl.ds(g * L, L)] for g in range(GROUPS)]

    NT = C // 128
    tile_used = []
    zeros = jnp.zeros((L,), jnp.int32)
    ones = jnp.ones((L,), jnp.int32)
    for t in range(NT):
        acc = zeros
        for g in range(GROUPS):
            acc = jnp.maximum(acc, jnp.where(colvs[g] >> 7 == t, ones, zeros))
        tile_used.append(jnp.max(acc) > 0)

    def din(ci, slot):
        for t in range(NT):
            @pl.when(tile_used[t])
            def _():
                pltpu.async_copy(
                    x_hbm.at[pl.ds(base + ci * CH, CH), pl.ds(128 * t, 128)],
                    inbufs[slot].at[pl.ds(0, CH), pl.ds(128 * t, 128)],
                    isems[slot],
                )

    def din_wait(slot):
        for t in range(NT):
            @pl.when(tile_used[t])
            def _():
                pltpu.make_async_copy(
                    x_hbm.at[pl.ds(0, CH), pl.ds(128 * t, 128)],
                    inbufs[slot].at[pl.ds(0, CH), pl.ds(128 * t, 128)],
                    isems[slot],
                ).wait()

    def dout(ci, slot):
        pltpu.async_copy(
            outbufs[slot], out_hbm.at[pl.ds(base + ci * CH, CH)], osems[slot]
        )

    def dout_wait(slot):
        pltpu.make_async_copy(
            outbufs[slot], out_hbm.at[pl.ds(0, CH)], osems[slot]
        ).wait()

    def compute(slot):
        ib, ob = inbufs[slot], outbufs[slot]

        @pl.loop(0, CH)
        def _row(r):
            rowv = jnp.full((L,), r, jnp.int32)
            for g in range(GROUPS):
                ob[r, pl.ds(g * L, L)] = plsc.load_gather(ib, [rowv, colvs[g]])

    din(0, 0)
    din(1, 1)

    @pl.loop(0, NCH // 2)
    def _pair(p):
        for slot in range(2):
            ci = p * 2 + slot
            din_wait(slot)

            @pl.when(p >= 1)
            def _():
                dout_wait(slot)

            compute(slot)
            dout(ci, slot)

            @pl.when(ci + 2 < NCH)
            def _():
                din(ci + 2, slot)

    dout_wait(0)
    dout_wait(1)


@jax.jit
def kernel(x, indices):
    xt = jnp.transpose(x, (2, 3, 0, 1)).reshape(PR, C)
    out = _lane_gather(xt, indices)
    return jnp.transpose(out.reshape(H, W, B, C), (2, 3, 0, 1))


# speedup vs baseline: 9.8542x; 2.6584x over previous
import functools

import jax
import jax.numpy as jnp
from jax import lax
from jax.experimental import pallas as pl
from jax.experimental.pallas import tpu as pltpu
from jax.experimental.pallas import tpu_sc as plsc

B, C, H, W = 64, 384, 28, 28
L = 16
NC, NS = 2, 16
NW = NC * NS
PR = H * W * B
RPW = PR // NW
CH = 56
NCH = RPW // CH
GROUPS = C // L

_mesh = plsc.VectorSubcoreMesh(
    core_axis_name="c", subcore_axis_name="s", num_cores=NC, num_subcores=NS
)


@functools.partial(
    pl.kernel,
    out_type=jax.ShapeDtypeStruct((PR, C), jnp.float32),
    mesh=_mesh,
    scratch_types=[
        pltpu.VMEM((C,), jnp.int32),
        [pltpu.VMEM((CH, C), jnp.float32)] * 2,
        [pltpu.VMEM((CH, C), jnp.float32)] * 2,
        [pltpu.SemaphoreType.DMA] * 2,
        [pltpu.SemaphoreType.DMA] * 2,
    ],
    compiler_params=pltpu.CompilerParams(needs_layout_passes=False),
)
def _lane_gather(x_hbm, idx_hbm, out_hbm, cidx_v, inbufs, outbufs, isems, osems):
    wid = lax.axis_index("s") * NC + lax.axis_index("c")
    base = wid * RPW

    pltpu.sync_copy(idx_hbm, cidx_v)
    colvs = [cidx_v[pl.ds(g * L, L)] for g in range(GROUPS)]

    NT = C // 128
    tile_used = []
    zeros = jnp.zeros((L,), jnp.int32)
    ones = jnp.ones((L,), jnp.int32)
    for t in range(NT):
        acc = zeros
        for g in range(GROUPS):
            acc = jnp.maximum(acc, jnp.where(colvs[g] >> 7 == t, ones, zeros))
        tile_used.append(jnp.max(acc) > 0)

    def din(ci, slot):
        for t in range(NT):
            @pl.when(tile_used[t])
            def _():
                pltpu.async_copy(
                    x_hbm.at[pl.ds(base + ci * CH, CH), pl.ds(128 * t, 128)],
                    inbufs[slot].at[pl.ds(0, CH), pl.ds(128 * t, 128)],
                    isems[slot],
                )

    def din_wait(slot):
        for t in range(NT):
            @pl.when(tile_used[t])
            def _():
                pltpu.make_async_copy(
                    x_hbm.at[pl.ds(0, CH), pl.ds(128 * t, 128)],
                    inbufs[slot].at[pl.ds(0, CH), pl.ds(128 * t, 128)],
                    isems[slot],
                ).wait()

    def dout(ci, slot):
        pltpu.async_copy(
            outbufs[slot], out_hbm.at[pl.ds(base + ci * CH, CH)], osems[slot]
        )

    def dout_wait(slot):
        pltpu.make_async_copy(
            outbufs[slot], out_hbm.at[pl.ds(0, CH)], osems[slot]
        ).wait()

    def compute(slot):
        ib, ob = inbufs[slot], outbufs[slot]

        @plsc.parallel_loop(0, CH, unroll=2)
        def _row(r):
            rowv = jnp.full((L,), r, jnp.int32)
            for g in range(GROUPS):
                ob[r, pl.ds(g * L, L)] = plsc.load_gather(ib, [rowv, colvs[g]])

    din(0, 0)
    din(1, 1)

    @pl.loop(0, NCH // 2)
    def _pair(p):
        for slot in range(2):
            ci = p * 2 + slot
            din_wait(slot)

            @pl.when(p >= 1)
            def _():
                dout_wait(slot)

            compute(slot)
            dout(ci, slot)

            @pl.when(ci + 2 < NCH)
            def _():
                din(ci + 2, slot)

    dout_wait(0)
    dout_wait(1)


@jax.jit
def kernel(x, indices):
    xt = jnp.transpose(x, (2, 3, 0, 1)).reshape(PR, C)
    out = _lane_gather(xt, indices)
    return jnp.transpose(out.reshape(H, W, B, C), (2, 3, 0, 1))
